# baseline (speedup 1.0000x reference)
import jax
import jax.numpy as jnp
from jax import lax
from jax.experimental import pallas as pl
from jax.experimental.pallas import tpu as pltpu
from jax.experimental.pallas import tpu_sc as plsc

N = 32768
D = 32
NSEG = 16
NW = 16
CHUNK = N // NW
L = 16
G = CHUNK // L
NEG = float("-inf")


def _sc_softmax(x_hbm, ids_hbm, w_hbm, out_hbm,
                xb, idsb, wb, ub, lrb, allb, gb, ob, tb, shmax, shsum):
    w = lax.axis_index("s")
    base = w * CHUNK
    pltpu.sync_copy(x_hbm.at[pl.ds(base * D, CHUNK * D)], xb)
    pltpu.sync_copy(ids_hbm.at[pl.ds(base, CHUNK)], idsb.at[pl.ds(0, CHUNK)])
    pltpu.sync_copy(w_hbm, wb)

    lanes = lax.iota(jnp.int32, L)
    idsb[pl.ds(CHUNK, L)] = jnp.full((L,), -1, jnp.int32)

    lane32 = lanes * D
    stripe = (lanes >> 2) << 3
    DB = 8
    for db in range(0, D, DB):
        wds_blk = [wb[pl.ds(d * L, L)] for d in range(db, db + DB)]
        bvs_blk = [lane32 + ((stripe + d) & (D - 1))
                   for d in range(db, db + DB)]

        def pass_a(g, carry, _first=(db == 0), _w=wds_blk, _b=bvs_blk):
            goff = g * (L * D)
            sl = pl.ds(g * L, L)
            acc = jnp.full((L,), 2.0 if _first else 0.0, jnp.float32)
            for j in range(DB):
                col = plsc.load_gather(xb, [_b[j] + goff])
                acc = acc + col * _w[j]
            if _first:
                ub[sl] = acc
            else:
                plsc.addupdate(ub.at[sl], acc)
            return carry

        lax.fori_loop(0, G, pass_a, 0)

    lrb[...] = jnp.full((L,), NEG, jnp.float32)
    lo = jnp.min(idsb[pl.ds(0, L)])
    hi = jnp.max(idsb[pl.ds(CHUNK - L, L)])

    def seg_max(s, carry):
        def g_body(g, macc):
            sl = pl.ds(g * L, L)
            return jnp.maximum(
                macc, jnp.where(idsb[sl] == s, ub[sl], NEG))
        macc = lax.fori_loop(0, G, g_body, jnp.full((L,), NEG, jnp.float32))
        m = jnp.max(macc)
        plsc.store_scatter(lrb, [jnp.full((L,), s, jnp.int32)],
                           jnp.full((L,), m, jnp.float32), mask=lanes == 0)
        return carry

    lax.fori_loop(lo, hi + 1, seg_max, 0)

    pltpu.sync_copy(lrb, shmax.at[pl.ds(w * NSEG, NSEG)])
    plsc.subcore_barrier()
    pltpu.sync_copy(shmax, allb)
    gm = allb[pl.ds(0, L)]
    for r in range(1, NW):
        gm = jnp.maximum(gm, allb[pl.ds(r * L, L)])
    gb[...] = gm

    lrb[...] = jnp.zeros((L,), jnp.float32)

    def pass_b(g, offs):
        sl = pl.ds(g * L, L)
        idsv = idsb[sl]
        mseg = plsc.load_gather(gb, [idsv])
        ev = jnp.exp(ub[sl] - mseg)
        ub[sl] = ev
        t = plsc.cumsum(ev) + offs
        bmask = idsv != idsb[pl.ds(g * L + 1, L)]
        plsc.store_scatter(lrb, [idsv], t, mask=bmask)
        return offs + jnp.sum(ev)

    lax.fori_loop(0, G, pass_b, jnp.zeros((L,), jnp.float32))

    v = lrb[...]
    fwd = plsc.cummax(v)
    tb[pl.ds(0, L)] = fwd
    prev = plsc.load_gather(tb, [jnp.maximum(lanes - 1, 0)])
    prev = jnp.where(lanes == 0, 0.0, prev)
    lrb[...] = jnp.where(v == 0.0, 0.0, v - prev)

    pltpu.sync_copy(lrb, shsum.at[pl.ds(w * NSEG, NSEG)])
    plsc.subcore_barrier()
    pltpu.sync_copy(shsum, allb)
    gs = allb[pl.ds(0, L)]
    for r in range(1, NW):
        gs = gs + allb[pl.ds(r * L, L)]
    gb[...] = 1.0 / gs

    def pass_c(g, carry):
        sl = pl.ds(g * L, L)
        rseg = plsc.load_gather(gb, [idsb[sl]])
        ob[sl] = ub[sl] * rseg
        return carry

    lax.fori_loop(0, G, pass_c, 0)
    pltpu.sync_copy(ob, out_hbm.at[pl.ds(base, CHUNK)])


def kernel(x, ids, W):
    ids32 = ids.astype(jnp.int32)
    xflat = x.reshape(N * D)
    ridx = (jnp.arange(D)[:, None] + 8 * (jnp.arange(L)[None, :] // 4)) % D
    wbc = W.reshape(-1)[ridx].reshape(D * L)
    mesh = plsc.VectorSubcoreMesh(
        core_axis_name="c", subcore_axis_name="s", num_cores=1,
        num_subcores=NW)
    soft = pl.kernel(
        _sc_softmax,
        out_type=jax.ShapeDtypeStruct((N,), jnp.float32),
        mesh=mesh,
        compiler_params=pltpu.CompilerParams(
            needs_layout_passes=False, disable_bounds_checks=True),
        scratch_types=[
            pltpu.VMEM((CHUNK * D,), jnp.float32),
            pltpu.VMEM((CHUNK + L,), jnp.int32),
            pltpu.VMEM((D * L,), jnp.float32),
            pltpu.VMEM((CHUNK,), jnp.float32),
            pltpu.VMEM((NSEG,), jnp.float32),
            pltpu.VMEM((NW * NSEG,), jnp.float32),
            pltpu.VMEM((NSEG,), jnp.float32),
            pltpu.VMEM((CHUNK,), jnp.float32),
            pltpu.VMEM((L,), jnp.float32),
            pltpu.VMEM_SHARED((NW * NSEG,), jnp.float32),
            pltpu.VMEM_SHARED((NW * NSEG,), jnp.float32),
        ],
    )(xflat, ids32, wbc)
    return soft[:, None]


# speedup vs baseline: 1.1376x; 1.1376x over previous
import jax
import jax.numpy as jnp
from jax import lax
from jax.experimental import pallas as pl
from jax.experimental.pallas import tpu as pltpu
from jax.experimental.pallas import tpu_sc as plsc

N = 32768
D = 32
NSEG = 16
NW = 16
CHUNK = N // NW
L = 16
G = CHUNK // L
NEG = float("-inf")


def _sc_softmax(x_hbm, ids_hbm, w_hbm, out_hbm,
                xb, idsb, wb, ub, lrb, allb, gb, ob, tb, shmax, shsum):
    w = lax.axis_index("s")
    base = w * CHUNK
    pltpu.sync_copy(x_hbm.at[pl.ds(base * D, CHUNK * D)], xb)
    pltpu.sync_copy(ids_hbm.at[pl.ds(base, CHUNK)], idsb.at[pl.ds(0, CHUNK)])
    pltpu.sync_copy(w_hbm, wb)

    lanes = lax.iota(jnp.int32, L)
    idsb[pl.ds(CHUNK, L)] = jnp.full((L,), -1, jnp.int32)

    lane32 = lanes * D
    stripe = (lanes >> 2) << 3
    DB = 8
    for db in range(0, D, DB):
        wds_blk = [wb[pl.ds(d * L, L)] for d in range(db, db + DB)]
        bvs_blk = [lane32 + ((stripe + d) & (D - 1))
                   for d in range(db, db + DB)]

        @plsc.parallel_loop(0, G, unroll=4)
        def pass_a(g, _first=(db == 0), _w=wds_blk, _b=bvs_blk):
            goff = g * (L * D)
            sl = pl.ds(g * L, L)
            ps = [plsc.load_gather(xb, [_b[j] + goff]) * _w[j]
                  for j in range(DB)]
            s1 = [ps[0] + ps[1], ps[2] + ps[3], ps[4] + ps[5], ps[6] + ps[7]]
            acc = (s1[0] + s1[1]) + (s1[2] + s1[3])
            if _first:
                ub[sl] = acc + 2.0
            else:
                plsc.addupdate(ub.at[sl], acc)

    lrb[...] = jnp.full((L,), NEG, jnp.float32)
    lo = jnp.min(idsb[pl.ds(0, L)])
    hi = jnp.max(idsb[pl.ds(CHUNK - L, L)])

    def seg_max(s, carry):
        def g_body(g, maccs):
            new = []
            for k in range(4):
                sl = pl.ds((4 * g + k) * L, L)
                new.append(jnp.maximum(
                    maccs[k], jnp.where(idsb[sl] == s, ub[sl], NEG)))
            return tuple(new)
        init = tuple(jnp.full((L,), NEG, jnp.float32) for _ in range(4))
        maccs = lax.fori_loop(0, G // 4, g_body, init)
        m = jnp.max(jnp.maximum(jnp.maximum(maccs[0], maccs[1]),
                                jnp.maximum(maccs[2], maccs[3])))
        plsc.store_scatter(lrb, [jnp.full((L,), s, jnp.int32)],
                           jnp.full((L,), m, jnp.float32), mask=lanes == 0)
        return carry

    lax.fori_loop(lo, hi + 1, seg_max, 0)

    pltpu.sync_copy(lrb, shmax.at[pl.ds(w * NSEG, NSEG)])
    plsc.subcore_barrier()
    pltpu.sync_copy(shmax, allb)
    gm = allb[pl.ds(0, L)]
    for r in range(1, NW):
        gm = jnp.maximum(gm, allb[pl.ds(r * L, L)])
    gb[...] = gm

    lrb[...] = jnp.zeros((L,), jnp.float32)

    def pass_b(g, offs):
        sl = pl.ds(g * L, L)
        idsv = idsb[sl]
        mseg = plsc.load_gather(gb, [idsv])
        ev = jnp.exp(ub[sl] - mseg)
        ub[sl] = ev
        t = plsc.cumsum(ev) + offs
        bmask = idsv != idsb[pl.ds(g * L + 1, L)]
        plsc.store_scatter(lrb, [idsv], t, mask=bmask)
        return offs + jnp.sum(ev)

    lax.fori_loop(0, G, pass_b, jnp.zeros((L,), jnp.float32))

    v = lrb[...]
    fwd = plsc.cummax(v)
    tb[pl.ds(0, L)] = fwd
    prev = plsc.load_gather(tb, [jnp.maximum(lanes - 1, 0)])
    prev = jnp.where(lanes == 0, 0.0, prev)
    lrb[...] = jnp.where(v == 0.0, 0.0, v - prev)

    pltpu.sync_copy(lrb, shsum.at[pl.ds(w * NSEG, NSEG)])
    plsc.subcore_barrier()
    pltpu.sync_copy(shsum, allb)
    gs = allb[pl.ds(0, L)]
    for r in range(1, NW):
        gs = gs + allb[pl.ds(r * L, L)]
    gb[...] = 1.0 / gs

    @plsc.parallel_loop(0, G, unroll=4)
    def pass_c(g):
        sl = pl.ds(g * L, L)
        rseg = plsc.load_gather(gb, [idsb[sl]])
        ob[sl] = ub[sl] * rseg
    pltpu.sync_copy(ob, out_hbm.at[pl.ds(base, CHUNK)])


def kernel(x, ids, W):
    ids32 = ids.astype(jnp.int32)
    xflat = x.reshape(N * D)
    ridx = (jnp.arange(D)[:, None] + 8 * (jnp.arange(L)[None, :] // 4)) % D
    wbc = W.reshape(-1)[ridx].reshape(D * L)
    mesh = plsc.VectorSubcoreMesh(
        core_axis_name="c", subcore_axis_name="s", num_cores=1,
        num_subcores=NW)
    soft = pl.kernel(
        _sc_softmax,
        out_type=jax.ShapeDtypeStruct((N,), jnp.float32),
        mesh=mesh,
        compiler_params=pltpu.CompilerParams(
            needs_layout_passes=False, disable_bounds_checks=True),
        scratch_types=[
            pltpu.VMEM((CHUNK * D,), jnp.float32),
            pltpu.VMEM((CHUNK + L,), jnp.int32),
            pltpu.VMEM((D * L,), jnp.float32),
            pltpu.VMEM((CHUNK,), jnp.float32),
            pltpu.VMEM((NSEG,), jnp.float32),
            pltpu.VMEM((NW * NSEG,), jnp.float32),
            pltpu.VMEM((NSEG,), jnp.float32),
            pltpu.VMEM((CHUNK,), jnp.float32),
            pltpu.VMEM((L,), jnp.float32),
            pltpu.VMEM_SHARED((NW * NSEG,), jnp.float32),
            pltpu.VMEM_SHARED((NW * NSEG,), jnp.float32),
        ],
    )(xflat, ids32, wbc)
    return soft[:, None]
